# Initial kernel scaffold; baseline (speedup 1.0000x reference)
import jax
import jax.numpy as jnp
from jax.experimental import pallas as pl


def kernel(input, embed_weight, proj_weight):
    raise NotImplementedError("write your pallas kernel here")


# speedup vs baseline: 1.4078x; 1.4078x over previous
import functools
import math

import jax
import jax.numpy as jnp
from jax import lax
from jax.experimental import pallas as pl
from jax.experimental.pallas import tpu as pltpu
from jax.experimental.pallas import tpu_sc as plsc


def _sc_gather(ntok, din, nw, nch, ch):
    per_w = nch * ch
    mesh = plsc.VectorSubcoreMesh(core_axis_name="c", subcore_axis_name="s")

    @functools.partial(
        pl.kernel,
        mesh=mesh,
        out_type=jax.ShapeDtypeStruct((ntok, din), jnp.float32),
        scratch_types=[
            pltpu.VMEM((nch, ch), jnp.int32),
            pltpu.VMEM((ch, din), jnp.float32),
            pltpu.VMEM((ch, din), jnp.float32),
            pltpu.SemaphoreType.DMA,
            pltpu.SemaphoreType.DMA,
        ],
    )
    def gather(idx_hbm, table_hbm, out_hbm, idx_v, buf0, buf1, sem0, sem1):
        info = plsc.get_sparse_core_info()
        wid = lax.axis_index("s") * info.num_cores + lax.axis_index("c")
        base = wid * per_w
        pltpu.sync_copy(idx_hbm.at[wid], idx_v)
        bufs = (buf0, buf1)
        sems = (sem0, sem1)
        cps = [None, None]
        cps[0] = pltpu.async_copy(table_hbm.at[idx_v.at[0]], buf0, sem0)
        for c in range(nch):
            nxt = c + 1
            if nxt < nch:
                cps[nxt % 2] = pltpu.async_copy(
                    table_hbm.at[idx_v.at[nxt]], bufs[nxt % 2], sems[nxt % 2])
            cps[c % 2].wait()
            pltpu.sync_copy(bufs[c % 2], out_hbm.at[pl.ds(base + c * ch, ch)])

    return gather


def _tc_project(bsz, seq, din, dout, bs, scale):
    def body(x_ref, w_ref, o_ref):
        w = w_ref[...]
        for bi in range(bsz):
            y = lax.dot_general(
                x_ref[bi], w,
                (((1,), (1,)), ((), ())),
                preferred_element_type=jnp.float32)
            o_ref[:, bi, :] = y * scale

    return pl.pallas_call(
        body,
        grid=(seq // bs,),
        in_specs=[
            pl.BlockSpec((bsz, bs, din), lambda i: (0, i, 0)),
            pl.BlockSpec((dout, din), lambda i: (0, 0)),
        ],
        out_specs=pl.BlockSpec((bs, bsz, dout), lambda i: (i, 0, 0)),
        out_shape=jax.ShapeDtypeStruct((seq, bsz, dout), jnp.float32),
    )


def kernel(input, embed_weight, proj_weight):
    bsz, seq = input.shape
    _, din = embed_weight.shape
    dout = proj_weight.shape[0]
    scale = math.sqrt(float(dout))
    ntok = bsz * seq

    nw = 32
    ch = 32
    per_w = ntok // nw
    nch = per_w // ch

    idx3 = input.reshape(nw, nch, ch)
    gathered = _sc_gather(ntok, din, nw, nch, ch)(idx3, embed_weight)
    x = gathered.reshape(bsz, seq, din)
    return _tc_project(bsz, seq, din, dout, 256, scale)(x, proj_weight)
